# baseline (speedup 1.0000x reference)
import jax
import jax.numpy as jnp
from jax.experimental import pallas as pl
from jax.experimental.pallas import tpu as pltpu


def _fold_weights_ky(weight_oihw):
    w = weight_oihw
    rows = jnp.stack([
        jnp.stack([w[:, :, 0], w[:, :, 1] + w[:, :, 2]], axis=0),
        jnp.stack([w[:, :, 0] + w[:, :, 1], w[:, :, 2]], axis=0),
    ], axis=0)
    return jnp.moveaxis(rows, -1, 2)


def _upconv_kernel(x_ref, top_ref, bot_ref, w_ref, b_ref, o_ref):
    i = pl.program_id(1)
    last = pl.num_programs(1) - 1
    C = x_ref.shape[1]
    W2 = top_ref.shape[2]
    TH = x_ref.shape[2] // W2

    x = x_ref[0]
    top = jnp.where(i == 0, 0.0, top_ref[0]).astype(x.dtype)
    bot = jnp.where(i == last, 0.0, bot_ref[0]).astype(x.dtype)
    U = jnp.concatenate([top, x, bot], axis=1)

    lane = jax.lax.broadcasted_iota(jnp.int32, (1, U.shape[1]), 1) % W2
    Um = jnp.where(lane == 0, 0.0,
                   jnp.concatenate([U[:, :1], U[:, :-1]], axis=1)).astype(x.dtype)
    Up = jnp.where(lane == W2 - 1, 0.0,
                   jnp.concatenate([U[:, 1:], U[:, -1:]], axis=1)).astype(x.dtype)
    variants = (Um, U, Up)

    accs = []
    for py in range(2):
        acc = None
        for a in range(2):
            dy = (a - 1) if py == 0 else a
            lo = (1 + dy) * W2
            for kx in range(3):
                strip = variants[kx][:, lo:lo + TH * W2]
                d = jnp.dot(w_ref[py, a, kx], strip,
                            preferred_element_type=jnp.float32)
                acc = d if acc is None else acc + d
        accs.append(acc)

    y0 = accs[0].reshape(C, TH, W2)
    y1 = accs[1].reshape(C, TH, W2)
    y = jnp.stack([y0, y1], axis=2).reshape(C, 2 * TH, W2)
    y = y + b_ref[...]
    o_ref[0] = y.astype(o_ref.dtype)


def kernel(x_nchw, weight, bias):
    N, C, H, W = x_nchw.shape
    W2 = 2 * W
    TH = H
    for cand in (32, 16, 8, 4, 2, 1):
        if H % cand == 0:
            TH = cand
            break

    xu = jnp.repeat(x_nchw, 2, axis=3).reshape(N, C, H * W2)
    wt = _fold_weights_ky(weight)
    bt = jnp.broadcast_to(bias[:, None, None], (C, 1, W2)).astype(jnp.float32)

    return pl.pallas_call(
        _upconv_kernel,
        out_shape=jax.ShapeDtypeStruct((N, C, 2 * H, W2), x_nchw.dtype),
        grid=(N, H // TH),
        in_specs=[
            pl.BlockSpec((1, C, TH * W2), lambda n, i: (n, 0, i)),
            pl.BlockSpec((1, C, W2), lambda n, i: (n, 0, jnp.maximum(i * TH - 1, 0))),
            pl.BlockSpec((1, C, W2), lambda n, i: (n, 0, jnp.minimum(i * TH + TH, H - 1))),
            pl.BlockSpec((2, 2, 3, C, C), lambda n, i: (0, 0, 0, 0, 0)),
            pl.BlockSpec((C, 1, W2), lambda n, i: (0, 0, 0)),
        ],
        out_specs=pl.BlockSpec((1, C, 2 * TH, W2), lambda n, i: (n, 0, i, 0)),
        compiler_params=pltpu.CompilerParams(
            dimension_semantics=("parallel", "parallel"),
            vmem_limit_bytes=64 * 1024 * 1024,
        ),
    )(xu, xu, xu, wt, bt)
